# Initial kernel scaffold; baseline (speedup 1.0000x reference)
import functools

import jax
import jax.numpy as jnp
from jax import lax
from jax.experimental import pallas as pl
from jax.experimental.pallas import tpu as pltpu
from jax.experimental.pallas import tpu_sc as plsc

N_EDGES = 800000
DIM = 64
NC = 2
NS = 16
NW = NC * NS
PER_W = N_EDGES // NW
CHUNK = 200
N_CHUNKS = PER_W // CHUNK


def _sc_body(src_p, dst_p, src_n, dst_n, emb_user, emb_item, rel,
             out_p, out_n, rel_v, idx_s, idx_d, u_rows, i_rows, out_v, sem):
    wid = lax.axis_index("s") * NC + lax.axis_index("c")

    pltpu.sync_copy(rel, rel_v)
    r0 = rel_v[pl.ds(0, 16)]
    r1 = rel_v[pl.ds(16, 16)]
    r2 = rel_v[pl.ds(32, 16)]
    r3 = rel_v[pl.ds(48, 16)]

    for src, dst, out in ((src_p, dst_p, out_p), (src_n, dst_n, out_n)):
        def chunk_body(j, carry, src=src, dst=dst, out=out):
            base = wid * PER_W + j * CHUNK
            pltpu.sync_copy(src.at[pl.ds(base, CHUNK)], idx_s)
            pltpu.sync_copy(dst.at[pl.ds(base, CHUNK)], idx_d)
            cp_u = pltpu.async_copy(emb_user.at[idx_s], u_rows, sem)
            cp_i = pltpu.async_copy(emb_item.at[idx_d], i_rows, sem)
            cp_u.wait()
            cp_i.wait()

            def edge_body(e, c):
                acc = u_rows[e, pl.ds(0, 16)] * (r0 * i_rows[e, pl.ds(0, 16)])
                acc = acc + u_rows[e, pl.ds(16, 16)] * (r1 * i_rows[e, pl.ds(16, 16)])
                acc = acc + u_rows[e, pl.ds(32, 16)] * (r2 * i_rows[e, pl.ds(32, 16)])
                acc = acc + u_rows[e, pl.ds(48, 16)] * (r3 * i_rows[e, pl.ds(48, 16)])
                out_v[e] = jnp.sum(acc)
                return c

            lax.fori_loop(0, CHUNK, edge_body, 0)
            pltpu.sync_copy(out_v, out.at[pl.ds(base, CHUNK)])
            return carry

        lax.fori_loop(0, N_CHUNKS, chunk_body, 0)


@jax.jit
def _dist_multi(src_p, dst_p, src_n, dst_n, emb_user, emb_item, rel):
    mesh = plsc.VectorSubcoreMesh(core_axis_name="c", subcore_axis_name="s")
    f = pl.kernel(
        _sc_body,
        out_type=(
            jax.ShapeDtypeStruct((N_EDGES,), jnp.float32),
            jax.ShapeDtypeStruct((N_EDGES,), jnp.float32),
        ),
        mesh=mesh,
        scratch_types=[
            pltpu.VMEM((DIM,), jnp.float32),
            pltpu.VMEM((CHUNK,), jnp.int32),
            pltpu.VMEM((CHUNK,), jnp.int32),
            pltpu.VMEM((CHUNK, DIM), jnp.float32),
            pltpu.VMEM((CHUNK, DIM), jnp.float32),
            pltpu.VMEM((CHUNK,), jnp.float32),
            pltpu.SemaphoreType.DMA,
        ],
    )
    return f(src_p, dst_p, src_n, dst_n, emb_user, emb_item, rel)


def kernel(emb_user, emb_item, rel_embedding, edge_pos, edge_neg):
    rel = rel_embedding.reshape(DIM)
    return _dist_multi(edge_pos[0], edge_pos[1], edge_neg[0], edge_neg[1],
                       emb_user, emb_item, rel)


# speedup vs baseline: 1.8413x; 1.8413x over previous
import functools

import jax
import jax.numpy as jnp
from jax import lax
from jax.experimental import pallas as pl
from jax.experimental.pallas import tpu as pltpu
from jax.experimental.pallas import tpu_sc as plsc

N_EDGES = 800000
DIM = 64
NC = 2
NS = 16
NW = NC * NS
PER_W = N_EDGES // NW
CHUNK = 200
N_CHUNKS = PER_W // CHUNK


N_BLOCKS = (CHUNK + 15) // 16
PAD = N_BLOCKS * 16


def _sc_body(src_p, dst_p, src_n, dst_n, emb_user, emb_item, rel,
             out_p, out_n, rel_v, idx_s, idx_d, u_rows, i_rows, out_v,
             sem):
    wid = lax.axis_index("s") * NC + lax.axis_index("c")

    pltpu.sync_copy(rel, rel_v)
    rel_regs = [rel_v[pl.ds(k * 16, 16)] for k in range(DIM // 16)]
    iota16 = lax.broadcasted_iota(jnp.int32, (16,), 0)

    ibase = iota16 * DIM
    if CHUNK > 128:
        splits = ((0, 128), (128, CHUNK - 128))
    else:
        splits = ((0, CHUNK),)

    for src, dst, out in ((src_p, dst_p, out_p), (src_n, dst_n, out_n)):
        def chunk_body(j, carry, src=src, dst=dst, out=out):
            base = wid * PER_W + j * CHUNK
            pltpu.sync_copy(src.at[pl.ds(base, CHUNK)], idx_s)
            pltpu.sync_copy(dst.at[pl.ds(base, CHUNK)], idx_d)
            cps = []
            for lo, ln in splits:
                isrc = idx_s if (lo, ln) == (0, CHUNK) else idx_s.at[pl.ds(lo, ln)]
                idst = idx_d if (lo, ln) == (0, CHUNK) else idx_d.at[pl.ds(lo, ln)]
                cps.append(pltpu.async_copy(
                    emb_user.at[isrc], u_rows.at[pl.ds(lo, ln)], sem))
                cps.append(pltpu.async_copy(
                    emb_item.at[idst], i_rows.at[pl.ds(lo, ln)], sem))
            for cp in cps:
                cp.wait()

            def block_body(b, c):
                e_idx = b * 16 + iota16
                score = jnp.zeros((16,), jnp.float32)
                for d in range(DIM):
                    dv = jnp.full((16,), d, dtype=jnp.int32)
                    u = plsc.load_gather(u_rows, [e_idx, dv])
                    iv = plsc.load_gather(i_rows, [e_idx, dv])
                    score = score + u * (iv * rel_regs[d // 16][d % 16])
                out_v[pl.ds(b * 16, 16)] = score
                return c

            lax.fori_loop(0, N_BLOCKS, block_body, 0)
            pltpu.sync_copy(out_v.at[pl.ds(0, CHUNK)], out.at[pl.ds(base, CHUNK)])
            return carry

        lax.fori_loop(0, N_CHUNKS, chunk_body, 0)


@jax.jit
def _dist_multi(src_p, dst_p, src_n, dst_n, emb_user, emb_item, rel):
    mesh = plsc.VectorSubcoreMesh(core_axis_name="c", subcore_axis_name="s",
                                  num_cores=NC, num_subcores=NS)
    f = pl.kernel(
        _sc_body,
        out_type=(
            jax.ShapeDtypeStruct((N_EDGES,), jnp.float32),
            jax.ShapeDtypeStruct((N_EDGES,), jnp.float32),
        ),
        mesh=mesh,
        scratch_types=[
            pltpu.VMEM((DIM,), jnp.float32),
            pltpu.VMEM((CHUNK,), jnp.int32),
            pltpu.VMEM((CHUNK,), jnp.int32),
            pltpu.VMEM((PAD, DIM), jnp.float32),
            pltpu.VMEM((PAD, DIM), jnp.float32),
            pltpu.VMEM((PAD,), jnp.float32),
            pltpu.SemaphoreType.DMA,
        ],
        compiler_params=pltpu.CompilerParams(needs_layout_passes=False,
                                             use_tc_tiling_on_sc=False),
    )
    return f(src_p, dst_p, src_n, dst_n, emb_user, emb_item, rel)


def kernel(emb_user, emb_item, rel_embedding, edge_pos, edge_neg):
    rel = rel_embedding.reshape(DIM)
    return _dist_multi(edge_pos[0], edge_pos[1], edge_neg[0], edge_neg[1],
                       emb_user, emb_item, rel)


# speedup vs baseline: 2.2479x; 1.2208x over previous
import functools

import jax
import jax.numpy as jnp
from jax import lax
from jax.experimental import pallas as pl
from jax.experimental.pallas import tpu as pltpu
from jax.experimental.pallas import tpu_sc as plsc

N_EDGES = 800000
DIM = 64
NC = 2
NS = 16
NW = NC * NS
PER_W = N_EDGES // NW
CHUNK = 200
N_CHUNKS = PER_W // CHUNK
N_PAIRS = (N_CHUNKS - 1) // 2
N_FULL_BLOCKS = CHUNK // 16
TAIL_OFF = CHUNK - 16
SPLITS = ((0, 128), (128, CHUNK - 128)) if CHUNK > 128 else ((0, CHUNK),)

assert N_CHUNKS % 2 == 1 and CHUNK % 8 == 0 and PER_W % CHUNK == 0


def _sc_body(src_p, dst_p, src_n, dst_n, emb_user, emb_item, rel,
             out_p, out_n,
             rel_v, idx_s, idx_d, u_a, i_a, u_b, i_b, out_all, sem_a, sem_b):
    wid = lax.axis_index("s") * NC + lax.axis_index("c")
    base = wid * PER_W

    pltpu.sync_copy(rel, rel_v)
    iota16 = lax.broadcasted_iota(jnp.int32, (16,), 0)

    def issue(j, u_buf, i_buf, sem):
        for lo, ln in SPLITS:
            pltpu.async_copy(emb_user.at[idx_s.at[pl.ds(j * CHUNK + lo, ln)]],
                             u_buf.at[pl.ds(lo, ln)], sem)
            pltpu.async_copy(emb_item.at[idx_d.at[pl.ds(j * CHUNK + lo, ln)]],
                             i_buf.at[pl.ds(lo, ln)], sem)

    def drain(j, u_buf, i_buf, sem):
        for lo, ln in SPLITS:
            pltpu.make_async_copy(
                emb_user.at[idx_s.at[pl.ds(j * CHUNK + lo, ln)]],
                u_buf.at[pl.ds(lo, ln)], sem).wait()
            pltpu.make_async_copy(
                emb_item.at[idx_d.at[pl.ds(j * CHUNK + lo, ln)]],
                i_buf.at[pl.ds(lo, ln)], sem).wait()

    def block_at(j, off, u_buf, i_buf):
        e_idx = off + iota16

        def dim_group(k, score):
            rk = rel_v[pl.ds(k * 16, 16)]
            dbase = k * 16
            for d in range(16):
                dv = jnp.full((16,), d, dtype=jnp.int32) + dbase
                u = plsc.load_gather(u_buf, [e_idx, dv])
                iv = plsc.load_gather(i_buf, [e_idx, dv])
                score = score + u * (iv * rk[d])
            return score

        score = lax.fori_loop(0, DIM // 16, dim_group,
                              jnp.zeros((16,), jnp.float32))
        out_all[pl.ds(j * CHUNK + off, 16)] = score

    def compute(j, u_buf, i_buf):
        def block_body(b, c):
            block_at(j, b * 16, u_buf, i_buf)
            return c
        lax.fori_loop(0, N_FULL_BLOCKS, block_body, 0)
        block_at(j, TAIL_OFF, u_buf, i_buf)

    for src, dst, out in ((src_p, dst_p, out_p), (src_n, dst_n, out_n)):
        pltpu.sync_copy(src.at[pl.ds(base, PER_W)], idx_s)
        pltpu.sync_copy(dst.at[pl.ds(base, PER_W)], idx_d)
        issue(0, u_a, i_a, sem_a)

        def pair_body(t, c):
            j0 = 2 * t
            issue(j0 + 1, u_b, i_b, sem_b)
            drain(j0, u_a, i_a, sem_a)
            compute(j0, u_a, i_a)
            issue(j0 + 2, u_a, i_a, sem_a)
            drain(j0 + 1, u_b, i_b, sem_b)
            compute(j0 + 1, u_b, i_b)
            return c

        lax.fori_loop(0, N_PAIRS, pair_body, 0)
        drain(N_CHUNKS - 1, u_a, i_a, sem_a)
        compute(N_CHUNKS - 1, u_a, i_a)
        pltpu.sync_copy(out_all.at[pl.ds(0, PER_W)], out.at[pl.ds(base, PER_W)])


@jax.jit
def _dist_multi(src_p, dst_p, src_n, dst_n, emb_user, emb_item, rel):
    mesh = plsc.VectorSubcoreMesh(core_axis_name="c", subcore_axis_name="s",
                                  num_cores=NC, num_subcores=NS)
    f = pl.kernel(
        _sc_body,
        out_type=(
            jax.ShapeDtypeStruct((N_EDGES,), jnp.float32),
            jax.ShapeDtypeStruct((N_EDGES,), jnp.float32),
        ),
        mesh=mesh,
        scratch_types=[
            pltpu.VMEM((DIM,), jnp.float32),
            pltpu.VMEM((PER_W,), jnp.int32),
            pltpu.VMEM((PER_W,), jnp.int32),
            pltpu.VMEM((CHUNK, DIM), jnp.float32),
            pltpu.VMEM((CHUNK, DIM), jnp.float32),
            pltpu.VMEM((CHUNK, DIM), jnp.float32),
            pltpu.VMEM((CHUNK, DIM), jnp.float32),
            pltpu.VMEM((PER_W,), jnp.float32),
            pltpu.SemaphoreType.DMA,
            pltpu.SemaphoreType.DMA,
        ],
        compiler_params=pltpu.CompilerParams(needs_layout_passes=False,
                                             use_tc_tiling_on_sc=False),
    )
    return f(src_p, dst_p, src_n, dst_n, emb_user, emb_item, rel)


def kernel(emb_user, emb_item, rel_embedding, edge_pos, edge_neg):
    rel = rel_embedding.reshape(DIM)
    return _dist_multi(edge_pos[0], edge_pos[1], edge_neg[0], edge_neg[1],
                       emb_user, emb_item, rel)


# speedup vs baseline: 9.7753x; 4.3487x over previous
import functools

import jax
import jax.numpy as jnp
from jax import lax
from jax.experimental import pallas as pl
from jax.experimental.pallas import tpu as pltpu
from jax.experimental.pallas import tpu_sc as plsc

N_EDGES = 800000
DIM = 64
NC = 2
NS = 16
NW = NC * NS
PER_W = N_EDGES // NW
CHUNK = 200
N_CHUNKS = PER_W // CHUNK
N_PAIRS = (N_CHUNKS - 1) // 2
N_FULL_BLOCKS = CHUNK // 16
TAIL_OFF = CHUNK - 16
SPLITS = ((0, 128), (128, CHUNK - 128)) if CHUNK > 128 else ((0, CHUNK),)

assert N_CHUNKS % 2 == 1 and CHUNK % 8 == 0 and PER_W % CHUNK == 0


def _sc_body(src_p, dst_p, src_n, dst_n, emb_user, emb_item, rel,
             out_p, out_n,
             rel_v, idx_s, idx_d, u_a, i_a, u_b, i_b, out_all, trans,
             sem_a, sem_b):
    wid = lax.axis_index("s") * NC + lax.axis_index("c")
    base = wid * PER_W

    pltpu.sync_copy(rel, rel_v)
    iota16 = lax.broadcasted_iota(jnp.int32, (16,), 0)

    def issue(j, u_buf, i_buf, sem):
        for lo, ln in SPLITS:
            pltpu.async_copy(emb_user.at[idx_s.at[pl.ds(j * CHUNK + lo, ln)]],
                             u_buf.at[pl.ds(lo, ln)], sem)
            pltpu.async_copy(emb_item.at[idx_d.at[pl.ds(j * CHUNK + lo, ln)]],
                             i_buf.at[pl.ds(lo, ln)], sem)

    def drain(j, u_buf, i_buf, sem):
        for lo, ln in SPLITS:
            pltpu.make_async_copy(
                emb_user.at[idx_s.at[pl.ds(j * CHUNK + lo, ln)]],
                u_buf.at[pl.ds(lo, ln)], sem).wait()
            pltpu.make_async_copy(
                emb_item.at[idx_d.at[pl.ds(j * CHUNK + lo, ln)]],
                i_buf.at[pl.ds(lo, ln)], sem).wait()

    rel_regs = [rel_v[pl.ds(k * 16, 16)] for k in range(DIM // 16)]
    iota17 = iota16 * 17

    def block_at(j, off, u_buf, i_buf):
        for e in range(16):
            acc = None
            for k in range(DIM // 16):
                u = u_buf[off + e, pl.ds(k * 16, 16)]
                iv = i_buf[off + e, pl.ds(k * 16, 16)]
                t = u * (rel_regs[k] * iv)
                acc = t if acc is None else acc + t
            plsc.store_scatter(trans, [iota17 + e], acc)
        score = trans[pl.ds(0, 16)]
        for l in range(1, 16):
            score = score + trans[pl.ds(l * 17, 16)]
        out_all[pl.ds(j * CHUNK + off, 16)] = score

    def compute(j, u_buf, i_buf):
        def block_body(b, c):
            block_at(j, b * 16, u_buf, i_buf)
            return c
        lax.fori_loop(0, N_FULL_BLOCKS, block_body, 0)
        block_at(j, TAIL_OFF, u_buf, i_buf)

    for src, dst, out in ((src_p, dst_p, out_p), (src_n, dst_n, out_n)):
        pltpu.sync_copy(src.at[pl.ds(base, PER_W)], idx_s)
        pltpu.sync_copy(dst.at[pl.ds(base, PER_W)], idx_d)
        issue(0, u_a, i_a, sem_a)

        def pair_body(t, c):
            j0 = 2 * t
            issue(j0 + 1, u_b, i_b, sem_b)
            drain(j0, u_a, i_a, sem_a)
            compute(j0, u_a, i_a)
            issue(j0 + 2, u_a, i_a, sem_a)
            drain(j0 + 1, u_b, i_b, sem_b)
            compute(j0 + 1, u_b, i_b)
            return c

        lax.fori_loop(0, N_PAIRS, pair_body, 0)
        drain(N_CHUNKS - 1, u_a, i_a, sem_a)
        compute(N_CHUNKS - 1, u_a, i_a)
        pltpu.sync_copy(out_all.at[pl.ds(0, PER_W)], out.at[pl.ds(base, PER_W)])


@jax.jit
def _dist_multi(src_p, dst_p, src_n, dst_n, emb_user, emb_item, rel):
    mesh = plsc.VectorSubcoreMesh(core_axis_name="c", subcore_axis_name="s",
                                  num_cores=NC, num_subcores=NS)
    f = pl.kernel(
        _sc_body,
        out_type=(
            jax.ShapeDtypeStruct((N_EDGES,), jnp.float32),
            jax.ShapeDtypeStruct((N_EDGES,), jnp.float32),
        ),
        mesh=mesh,
        scratch_types=[
            pltpu.VMEM((DIM,), jnp.float32),
            pltpu.VMEM((PER_W,), jnp.int32),
            pltpu.VMEM((PER_W,), jnp.int32),
            pltpu.VMEM((CHUNK, DIM), jnp.float32),
            pltpu.VMEM((CHUNK, DIM), jnp.float32),
            pltpu.VMEM((CHUNK, DIM), jnp.float32),
            pltpu.VMEM((CHUNK, DIM), jnp.float32),
            pltpu.VMEM((PER_W,), jnp.float32),
            pltpu.VMEM((272,), jnp.float32),
            pltpu.SemaphoreType.DMA,
            pltpu.SemaphoreType.DMA,
        ],
        compiler_params=pltpu.CompilerParams(needs_layout_passes=False,
                                             use_tc_tiling_on_sc=False),
    )
    return f(src_p, dst_p, src_n, dst_n, emb_user, emb_item, rel)


def kernel(emb_user, emb_item, rel_embedding, edge_pos, edge_neg):
    rel = rel_embedding.reshape(DIM)
    return _dist_multi(edge_pos[0], edge_pos[1], edge_neg[0], edge_neg[1],
                       emb_user, emb_item, rel)
